# baseline (speedup 1.0000x reference)
import functools

import jax
import jax.numpy as jnp
from jax import lax
from jax.experimental import pallas as pl
from jax.experimental.pallas import tpu as pltpu
from jax.experimental.pallas import tpu_sc as plsc

DIM_WORD = 128
DIM_POS = 32
TOTAL_DIM = DIM_WORD + DIM_POS
LN_EPS = 1e-5
LANES = 16
CHUNK = 128


def _rsqrt16(x):
    bits = lax.bitcast_convert_type(x, jnp.int32)
    magic = jnp.full((LANES,), 0x5F3759DF, dtype=jnp.int32)
    y = lax.bitcast_convert_type(
        magic - lax.shift_right_logical(bits, 1), jnp.float32)
    half = x * 0.5
    for _ in range(2):
        y = y * (1.5 - half * y * y)
    return y


def _tree_sum(vals):
    vals = list(vals)
    while len(vals) > 1:
        vals = ([vals[i] + vals[i + 1] for i in range(0, len(vals) - 1, 2)]
                + ([vals[-1]] if len(vals) % 2 else []))
    return vals[0]


def _perm(x, idx):
    return x.at[idx].get(mode="promise_in_bounds")


def _splat(c):
    return jnp.full((LANES,), c, dtype=jnp.int32)


def _hsum_multi(ts, lane):
    vecs = list(ts)
    k = 0
    while len(vecs) > 1:
        sh = 1 << k
        m = (lane & sh) == 0
        permk = lax.bitwise_xor(lane, sh)
        nxt = []
        for i in range(0, len(vecs), 2):
            a2 = vecs[i] + _perm(vecs[i], permk)
            b2 = vecs[i + 1] + _perm(vecs[i + 1], permk)
            nxt.append(jnp.where(m, a2, b2))
        vecs = nxt
        k += 1
    v = vecs[0]
    while (1 << k) < LANES:
        v = v + _perm(v, lax.bitwise_xor(lane, 1 << k))
        k += 1
    return v


def _make_kernel(n_rows):
    info = plsc.get_sparse_core_info()
    num_workers = info.num_cores * info.num_subcores
    rows_per_worker = n_rows // num_workers
    num_chunks = rows_per_worker // CHUNK
    assert rows_per_worker * num_workers == n_rows
    assert num_chunks * CHUNK == rows_per_worker
    assert num_chunks % 2 == 0
    groups = CHUNK // LANES

    mesh = plsc.VectorSubcoreMesh(core_axis_name="c", subcore_axis_name="s")

    @functools.partial(
        pl.kernel,
        out_type=jax.ShapeDtypeStruct((n_rows * TOTAL_DIM,), jnp.float32),
        mesh=mesh,
        scratch_types=[
            [pltpu.VMEM((CHUNK,), jnp.int32)] * 2,
            [pltpu.VMEM((CHUNK,), jnp.int32)] * 2,
            [pltpu.VMEM((CHUNK, DIM_WORD // 2), jnp.int32)] * 2,
            pltpu.VMEM((64, DIM_POS), jnp.float32),
            [pltpu.VMEM((CHUNK * TOTAL_DIM,), jnp.float32)] * 2,
            pltpu.VMEM((TOTAL_DIM,), jnp.float32),
            pltpu.VMEM((TOTAL_DIM,), jnp.float32),
            [pltpu.SemaphoreType.DMA] * 2,
            [pltpu.SemaphoreType.DMA] * 2,
            [pltpu.SemaphoreType.DMA] * 2,
        ],
        compiler_params=pltpu.CompilerParams(needs_layout_passes=False,
                                             use_tc_tiling_on_sc=False),
    )
    def sc_kernel(words_hbm, postags_hbm, wtab_hbm, ptab_hbm, gamma_hbm,
                  beta_hbm, out_hbm, widx_v, pidx_v, wrows_v, ptab_v,
                  out_v, gamma_v, beta_v, sem_i, sem_w, sem_o):
        wid = lax.axis_index("s") * info.num_cores + lax.axis_index("c")
        base = wid * rows_per_worker
        lane = lax.iota(jnp.int32, LANES)

        pltpu.sync_copy(ptab_hbm, ptab_v)
        pltpu.sync_copy(gamma_hbm, gamma_v)
        pltpu.sync_copy(beta_hbm, beta_v)
        gvs = [gamma_v[pl.ds(k * LANES, LANES)] for k in range(TOTAL_DIM // LANES)]
        bvs = [beta_v[pl.ds(k * LANES, LANES)] for k in range(TOTAL_DIM // LANES)]

        def idx_start(c, b):
            row0 = base + c * CHUNK
            pltpu.async_copy(words_hbm.at[pl.ds(row0, CHUNK)], widx_v[b],
                             sem_i[b])
            pltpu.async_copy(postags_hbm.at[pl.ds(row0, CHUNK)], pidx_v[b],
                             sem_i[b])

        def idx_wait(b):
            pltpu.make_async_copy(words_hbm.at[pl.ds(0, CHUNK)], widx_v[b],
                                  sem_i[b]).wait()
            pltpu.make_async_copy(postags_hbm.at[pl.ds(0, CHUNK)], pidx_v[b],
                                  sem_i[b]).wait()

        def gather_start(b):
            pltpu.async_copy(wtab_hbm.at[widx_v[b]], wrows_v[b], sem_w[b])

        def gather_wait(b):
            pltpu.make_async_copy(wtab_hbm.at[pl.ds(0, CHUNK)], wrows_v[b],
                                  sem_w[b]).wait()

        def wb_start(c, b):
            e0 = (base + c * CHUNK) * TOTAL_DIM
            pltpu.async_copy(out_v[b], out_hbm.at[pl.ds(e0, CHUNK * TOTAL_DIM)],
                             sem_o[b])

        def wb_wait(b):
            pltpu.make_async_copy(out_v[b],
                                  out_hbm.at[pl.ds(0, CHUNK * TOTAL_DIM)],
                                  sem_o[b]).wait()

        def compute(b):
            def load_row(pvec, rb, j):
                r = rb + j
                p = pvec[j]
                vs = []
                himask = jnp.full((LANES,), -65536, dtype=jnp.int32)
                for k in range(DIM_WORD // 32):
                    w = wrows_v[b][r, pl.ds(k * LANES, LANES)]
                    lo = lax.bitcast_convert_type(
                        lax.shift_left(w, 16), jnp.float32)
                    hi = lax.bitcast_convert_type(
                        lax.bitwise_and(w, himask), jnp.float32)
                    vs += [lo, hi]
                vs += [ptab_v[p, pl.ds(k * LANES, LANES)]
                       for k in range(DIM_POS // LANES)]
                return vs

            def group_body(g, carry):
                rb = g * LANES
                pvec = pidx_v[b][pl.ds(rb, LANES)]
                tsums = []
                qsums = []
                for j in range(LANES):
                    vs = load_row(pvec, rb, j)
                    tsums.append(_tree_sum(vs))
                    qsums.append(_tree_sum([v * v for v in vs]))
                s = _hsum_multi(tsums, lane)
                q = _hsum_multi(qsums, lane)
                mean = s * (1.0 / TOTAL_DIM)
                var = q * (1.0 / TOTAL_DIM) - mean * mean
                inv = _rsqrt16(var + LN_EPS)
                for j in range(LANES):
                    jj = _splat(j)
                    m_j = _perm(mean, jj)
                    i_j = _perm(inv, jj)
                    r = rb + j
                    vs = load_row(pvec, rb, j)
                    e0 = r * TOTAL_DIM
                    for k, v in enumerate(vs):
                        sl = pl.ds(e0 + k * LANES, LANES)
                        out_v[b][sl] = (v - m_j) * i_j * gvs[k] + bvs[k]
                return carry

            lax.fori_loop(0, groups, group_body, 0, unroll=False)

        idx_start(0, 0)
        idx_start(1, 1)
        idx_wait(0)
        gather_start(0)

        def pair_body(i, carry):
            for phase in range(2):
                c = 2 * i + phase
                b = phase
                nb = 1 - phase

                @pl.when(c + 1 < num_chunks)
                def _():
                    idx_wait(nb)
                    gather_start(nb)

                gather_wait(b)

                @pl.when(c >= 2)
                def _():
                    wb_wait(b)

                compute(b)

                @pl.when(c + 2 < num_chunks)
                def _():
                    idx_start(c + 2, b)

                wb_start(c, b)
            return carry

        lax.fori_loop(0, num_chunks // 2, pair_body, 0, unroll=False)
        wb_wait(0)
        wb_wait(1)

    return sc_kernel


def kernel(words, postags, word_table, pos_table, ln_gamma, ln_beta):
    b, l = words.shape
    n = b * l
    v = word_table.shape[0]
    wt = (word_table.reshape(v, DIM_WORD // 32, 2, LANES)
          .swapaxes(2, 3)
          .astype(jnp.bfloat16)
          .reshape(v, DIM_WORD // 2, 2))
    wt = lax.bitcast_convert_type(wt, jnp.int32)
    out = _make_kernel(n)(
        words.reshape(n), postags.reshape(n), wt, pos_table,
        ln_gamma, ln_beta)
    return out.reshape(b, l, TOTAL_DIM)


# speedup vs baseline: 1.3219x; 1.3219x over previous
import functools

import jax
import jax.numpy as jnp
from jax import lax
from jax.experimental import pallas as pl
from jax.experimental.pallas import tpu as pltpu
from jax.experimental.pallas import tpu_sc as plsc

DIM_WORD = 128
DIM_POS = 32
TOTAL_DIM = DIM_WORD + DIM_POS
LN_EPS = 1e-5
LANES = 16
CHUNK = 128


def _rsqrt16(x):
    bits = lax.bitcast_convert_type(x, jnp.int32)
    magic = jnp.full((LANES,), 0x5F3759DF, dtype=jnp.int32)
    y = lax.bitcast_convert_type(
        magic - lax.shift_right_logical(bits, 1), jnp.float32)
    half = x * 0.5
    for _ in range(2):
        y = y * (1.5 - half * y * y)
    return y


def _tree_sum(vals):
    vals = list(vals)
    while len(vals) > 1:
        vals = ([vals[i] + vals[i + 1] for i in range(0, len(vals) - 1, 2)]
                + ([vals[-1]] if len(vals) % 2 else []))
    return vals[0]


def _perm(x, idx):
    return x.at[idx].get(mode="promise_in_bounds")


def _splat(c):
    return jnp.full((LANES,), c, dtype=jnp.int32)


def _hsum_multi(ts, lane):
    vecs = list(ts)
    k = 0
    while len(vecs) > 1:
        sh = 1 << k
        m = (lane & sh) == 0
        permk = lax.bitwise_xor(lane, sh)
        nxt = []
        for i in range(0, len(vecs), 2):
            a2 = vecs[i] + _perm(vecs[i], permk)
            b2 = vecs[i + 1] + _perm(vecs[i + 1], permk)
            nxt.append(jnp.where(m, a2, b2))
        vecs = nxt
        k += 1
    v = vecs[0]
    while (1 << k) < LANES:
        v = v + _perm(v, lax.bitwise_xor(lane, 1 << k))
        k += 1
    return v


def _make_kernel(n_rows):
    info = plsc.get_sparse_core_info()
    num_workers = info.num_cores * info.num_subcores
    rows_per_worker = n_rows // num_workers
    num_chunks = rows_per_worker // CHUNK
    assert rows_per_worker * num_workers == n_rows
    assert num_chunks * CHUNK == rows_per_worker
    assert num_chunks % 2 == 0
    groups = CHUNK // LANES

    mesh = plsc.VectorSubcoreMesh(core_axis_name="c", subcore_axis_name="s")

    @functools.partial(
        pl.kernel,
        out_type=jax.ShapeDtypeStruct((n_rows * TOTAL_DIM,), jnp.float32),
        mesh=mesh,
        scratch_types=[
            [pltpu.VMEM((CHUNK,), jnp.int32)] * 2,
            [pltpu.VMEM((CHUNK,), jnp.int32)] * 2,
            [pltpu.VMEM((CHUNK, DIM_WORD), jnp.float32)] * 2,
            pltpu.VMEM((64, DIM_POS), jnp.float32),
            [pltpu.VMEM((CHUNK * TOTAL_DIM,), jnp.float32)] * 2,
            pltpu.VMEM((TOTAL_DIM,), jnp.float32),
            pltpu.VMEM((TOTAL_DIM,), jnp.float32),
            [pltpu.SemaphoreType.DMA] * 2,
            [pltpu.SemaphoreType.DMA] * 2,
            [pltpu.SemaphoreType.DMA] * 2,
        ],
    )
    def sc_kernel(words_hbm, postags_hbm, wtab_hbm, ptab_hbm, gamma_hbm,
                  beta_hbm, out_hbm, widx_v, pidx_v, wrows_v, ptab_v,
                  out_v, gamma_v, beta_v, sem_i, sem_w, sem_o):
        wid = lax.axis_index("s") * info.num_cores + lax.axis_index("c")
        base = wid * rows_per_worker
        lane = lax.iota(jnp.int32, LANES)

        pltpu.sync_copy(ptab_hbm, ptab_v)
        pltpu.sync_copy(gamma_hbm, gamma_v)
        pltpu.sync_copy(beta_hbm, beta_v)
        gvs = [gamma_v[pl.ds(k * LANES, LANES)] for k in range(TOTAL_DIM // LANES)]
        bvs = [beta_v[pl.ds(k * LANES, LANES)] for k in range(TOTAL_DIM // LANES)]

        def idx_start(c, b):
            row0 = base + c * CHUNK
            pltpu.async_copy(words_hbm.at[pl.ds(row0, CHUNK)],
                             widx_v[b], sem_i[b])
            pltpu.async_copy(postags_hbm.at[pl.ds(row0, CHUNK)], pidx_v[b],
                             sem_i[b])

        def idx_wait(b):
            pltpu.make_async_copy(words_hbm.at[pl.ds(0, CHUNK)],
                                  widx_v[b], sem_i[b]).wait()
            pltpu.make_async_copy(postags_hbm.at[pl.ds(0, CHUNK)], pidx_v[b],
                                  sem_i[b]).wait()

        def gather_start(b):
            pltpu.async_copy(wtab_hbm.at[widx_v[b]], wrows_v[b], sem_w[b])

        def gather_wait(b):
            pltpu.make_async_copy(wtab_hbm.at[widx_v[b]], wrows_v[b],
                                  sem_w[b]).wait()

        def wb_start(c, b):
            e0 = (base + c * CHUNK) * TOTAL_DIM
            pltpu.async_copy(out_v[b], out_hbm.at[pl.ds(e0, CHUNK * TOTAL_DIM)],
                             sem_o[b])

        def wb_wait(b):
            pltpu.make_async_copy(out_v[b],
                                  out_hbm.at[pl.ds(0, CHUNK * TOTAL_DIM)],
                                  sem_o[b]).wait()

        def compute(b):
            def load_row(pvec, rb, j):
                r = rb + j
                p = pvec[j]
                vs = [wrows_v[b][r, pl.ds(k * LANES, LANES)]
                      for k in range(DIM_WORD // LANES)]
                vs += [ptab_v[p, pl.ds(k * LANES, LANES)]
                       for k in range(DIM_POS // LANES)]
                return vs

            def group_body(g, carry):
                rb = g * LANES
                pvec = pidx_v[b][pl.ds(rb, LANES)]
                tsums = []
                qsums = []
                for j in range(LANES):
                    vs = load_row(pvec, rb, j)
                    tsums.append(_tree_sum(vs))
                    qsums.append(_tree_sum([v * v for v in vs]))
                s = _hsum_multi(tsums, lane)
                q = _hsum_multi(qsums, lane)
                mean = s * (1.0 / TOTAL_DIM)
                var = q * (1.0 / TOTAL_DIM) - mean * mean
                inv = _rsqrt16(var + LN_EPS)
                for j in range(LANES):
                    jj = _splat(j)
                    m_j = _perm(mean, jj)
                    i_j = _perm(inv, jj)
                    r = rb + j
                    vs = load_row(pvec, rb, j)
                    e0 = r * TOTAL_DIM
                    for k, v in enumerate(vs):
                        sl = pl.ds(e0 + k * LANES, LANES)
                        out_v[b][sl] = (v - m_j) * i_j * gvs[k] + bvs[k]
                return carry

            lax.fori_loop(0, groups, group_body, 0, unroll=False)

        idx_start(0, 0)
        idx_start(1, 1)
        idx_wait(0)
        gather_start(0)

        def pair_body(i, carry):
            for phase in range(2):
                c = 2 * i + phase
                b = phase
                nb = 1 - phase

                @pl.when(c + 1 < num_chunks)
                def _():
                    idx_wait(nb)
                    gather_start(nb)

                gather_wait(b)

                @pl.when(c >= 2)
                def _():
                    wb_wait(b)

                compute(b)

                @pl.when(c + 2 < num_chunks)
                def _():
                    idx_start(c + 2, b)

                wb_start(c, b)
            return carry

        lax.fori_loop(0, num_chunks // 2, pair_body, 0, unroll=False)
        wb_wait(0)
        wb_wait(1)

    return sc_kernel


def kernel(words, postags, word_table, pos_table, ln_gamma, ln_beta):
    b, l = words.shape
    n = b * l
    out = _make_kernel(n)(
        words.reshape(n), postags.reshape(n), word_table, pos_table,
        ln_gamma, ln_beta)
    return out.reshape(b, l, TOTAL_DIM)
